# baseline (speedup 1.0000x reference)
import functools

import jax
import jax.numpy as jnp
from jax import lax
from jax.experimental import pallas as pl
from jax.experimental.pallas import tpu as pltpu
from jax.experimental.pallas import tpu_sc as plsc

_NUM_ROWS = 1025
_DIM = 768


def kernel(table):
    info = plsc.get_sparse_core_info()
    nc, ns = info.num_cores, info.num_subcores
    nw = nc * ns
    rows_per_w = _NUM_ROWS // nw
    rem = _NUM_ROWS - rows_per_w * nw

    mesh = plsc.VectorSubcoreMesh(core_axis_name="c", subcore_axis_name="s")

    @functools.partial(
        pl.kernel,
        mesh=mesh,
        out_type=jax.ShapeDtypeStruct((1, _NUM_ROWS, _DIM), jnp.float32),
    )
    def copy_rows(table_hbm, out_hbm):
        wid = lax.axis_index("s") * nc + lax.axis_index("c")
        base = wid * rows_per_w
        pltpu.sync_copy(
            table_hbm.at[pl.ds(base, rows_per_w)],
            out_hbm.at[0, pl.ds(base, rows_per_w)],
        )

        @pl.when(wid == 0)
        def _copy_tail():
            tail = nw * rows_per_w
            pltpu.sync_copy(
                table_hbm.at[pl.ds(tail, rem)],
                out_hbm.at[0, pl.ds(tail, rem)],
            )

    return copy_rows(table)


# speedup vs baseline: 3.9334x; 3.9334x over previous
import functools

import jax
import jax.numpy as jnp
from jax import lax
from jax.experimental import pallas as pl
from jax.experimental.pallas import tpu as pltpu
from jax.experimental.pallas import tpu_sc as plsc

_NUM_ROWS = 1025
_DIM = 768


def kernel(table):
    info = plsc.get_sparse_core_info()
    nc, ns = info.num_cores, info.num_subcores
    nw = nc * ns
    rows_per_w = _NUM_ROWS // nw
    rem = _NUM_ROWS - rows_per_w * nw

    mesh = plsc.VectorSubcoreMesh(core_axis_name="c", subcore_axis_name="s")

    @functools.partial(
        pl.kernel,
        mesh=mesh,
        out_type=jax.ShapeDtypeStruct((1, _NUM_ROWS, _DIM), jnp.float32),
        scratch_types=[
            pltpu.VMEM((rows_per_w, _DIM), jnp.float32),
            pltpu.VMEM((rem, _DIM), jnp.float32),
        ],
    )
    def copy_rows(table_hbm, out_hbm, buf, tail_buf):
        wid = lax.axis_index("s") * nc + lax.axis_index("c")
        base = wid * rows_per_w
        pltpu.sync_copy(table_hbm.at[pl.ds(base, rows_per_w)], buf)
        pltpu.sync_copy(buf, out_hbm.at[0, pl.ds(base, rows_per_w)])

        @pl.when(wid == 0)
        def _copy_tail():
            tail = nw * rows_per_w
            pltpu.sync_copy(table_hbm.at[pl.ds(tail, rem)], tail_buf)
            pltpu.sync_copy(tail_buf, out_hbm.at[0, pl.ds(tail, rem)])

    return copy_rows(table)
